# Initial kernel scaffold; baseline (speedup 1.0000x reference)
import jax
import jax.numpy as jnp
from jax.experimental import pallas as pl


def kernel(x, edge, batch, gcn_W0, gcn_b0, gcn_W1, gcn_b1, sage_Wn0, sage_Wr0, sage_b0, sage_Wn1, sage_Wr1, sage_b1, gat_W0, gat_as0, gat_ad0, gat_b0, gat_W1, gat_as1, gat_ad1, gat_b1):
    raise NotImplementedError("write your pallas kernel here")


# speedup vs baseline: 22.1852x; 22.1852x over previous
import functools

import jax
import jax.numpy as jnp
from jax import lax
from jax.experimental import pallas as pl
from jax.experimental.pallas import tpu as pltpu
from jax.experimental.pallas import tpu_sc as plsc

_N = 10000
_E = 320000
_D = 128
_NC = 2
_NS = 16
_NW = _NC * _NS
_EPT = _E // _NW
_CH = 80
_NCHUNK = _EPT // _CH
_RPT = _N // _NS
_BN = 1000
_G = _N // _BN

_mesh = plsc.VectorSubcoreMesh(core_axis_name="c", subcore_axis_name="s")
_f32 = jnp.float32


def _worker_ids():
    c = lax.axis_index("c")
    s = lax.axis_index("s")
    return c, s, s * _NC + c


def _zero_vec(ref, n):
    def body(i, _):
        ref[pl.ds(i * 16, 16)] = jnp.zeros((16,), _f32)
        return 0
    lax.fori_loop(0, n // 16, body, 0)


def _zero_acc(zbuf, acc, s):
    def body(i, _):
        for k in range(_D // 16):
            zbuf[i, pl.ds(k * 16, 16)] = jnp.zeros((16,), _f32)
        return 0
    lax.fori_loop(0, 125, body, 0)
    rbase = s * _RPT
    for k in range(5):
        pltpu.sync_copy(zbuf, acc.at[pl.ds(rbase + k * 125, 125)])


@functools.partial(
    pl.kernel,
    out_type=jax.ShapeDtypeStruct((_G, _NW, _BN), _f32),
    mesh=_mesh,
    compiler_params=pltpu.CompilerParams(needs_layout_passes=False, use_tc_tiling_on_sc=False),
    scratch_types=[
        pltpu.VMEM((_EPT,), jnp.int32),
        pltpu.VMEM((_N,), _f32),
    ],
)
def _sc_degree(dst_hbm, cntp_hbm, dst_v, cnt_v):
    _, _, wid = _worker_ids()
    base = wid * _EPT
    pltpu.sync_copy(dst_hbm.at[pl.ds(base, _EPT)], dst_v)
    _zero_vec(cnt_v, _N)
    ones = jnp.ones((16,), _f32)

    def body(i, _):
        d16 = dst_v[pl.ds(i * 16, 16)]
        plsc.addupdate_scatter(cnt_v, [d16], ones)
        return 0
    lax.fori_loop(0, _EPT // 16, body, 0)
    for g in range(_G):
        pltpu.sync_copy(cnt_v.at[pl.ds(g * _BN, _BN)], cntp_hbm.at[g, wid])


@functools.partial(
    pl.kernel,
    out_type=jax.ShapeDtypeStruct((_NC, _N, _D), _f32),
    mesh=_mesh,
    compiler_params=pltpu.CompilerParams(needs_layout_passes=False, use_tc_tiling_on_sc=False),
    scratch_types=[
        pltpu.VMEM((_CH,), jnp.int32),
        pltpu.VMEM((_CH,), jnp.int32),
        pltpu.VMEM((_CH, _D), _f32),
        pltpu.VMEM((125, _D), _f32),
        pltpu.VMEM_SHARED((_N, _D), _f32),
        pltpu.SemaphoreType.DMA,
    ],
)
def _sc_segsum(src_hbm, dst_hbm, tab_hbm, out_hbm, sidx, didx, rows, zbuf,
               acc, sem):
    c, s, wid = _worker_ids()
    base = wid * _EPT
    _zero_acc(zbuf, acc, s)
    plsc.subcore_barrier()

    def body(i, _):
        eb = base + i * _CH
        pltpu.sync_copy(src_hbm.at[pl.ds(eb, _CH)], sidx)
        pltpu.sync_copy(dst_hbm.at[pl.ds(eb, _CH)], didx)
        pltpu.async_copy(tab_hbm.at[sidx], rows, sem).wait()
        pltpu.sync_copy(rows, acc.at[didx], add=True)
        return 0
    lax.fori_loop(0, _NCHUNK, body, 0)
    plsc.subcore_barrier()
    rbase = s * _RPT
    pltpu.sync_copy(acc.at[pl.ds(rbase, _RPT)],
                    out_hbm.at[c, pl.ds(rbase, _RPT)])


def _make_gat_edge(H):
    @functools.partial(
        pl.kernel,
        out_type=(jax.ShapeDtypeStruct((H, _E), _f32),
                  jax.ShapeDtypeStruct((_G, _NW, H, _BN), _f32)),
        mesh=_mesh,
        compiler_params=pltpu.CompilerParams(needs_layout_passes=False, use_tc_tiling_on_sc=False),
        scratch_types=[
            pltpu.VMEM((_EPT,), jnp.int32),
            pltpu.VMEM((_EPT,), jnp.int32),
            pltpu.VMEM((_N,), _f32),
            pltpu.VMEM((_N,), _f32),
            pltpu.VMEM((_N,), _f32),
            pltpu.VMEM((_EPT,), _f32),
            pltpu.VMEM((H, 128), _f32),
            pltpu.VMEM((H, 128), _f32),
        ],
    )
    def _sc_gat_edge(src_hbm, dst_hbm, asrcT, adstT, sm_hbm, am_hbm,
                     ex_hbm, denp_hbm,
                     src_v, dst_v, as_v, ad_v, den_v, ex_v, sm_v, am_v):
        _, _, wid = _worker_ids()
        base = wid * _EPT
        pltpu.sync_copy(src_hbm.at[pl.ds(base, _EPT)], src_v)
        pltpu.sync_copy(dst_hbm.at[pl.ds(base, _EPT)], dst_v)
        pltpu.sync_copy(sm_hbm, sm_v)
        pltpu.sync_copy(am_hbm, am_v)
        for h in range(H):
            for g in range(_G):
                pltpu.sync_copy(asrcT.at[g, h], as_v.at[pl.ds(g * _BN, _BN)])
                pltpu.sync_copy(adstT.at[g, h], ad_v.at[pl.ds(g * _BN, _BN)])
            sv = sm_v[h, pl.ds(0, 16)]
            av = am_v[h, pl.ds(0, 16)]
            m = jnp.maximum(sv[0] + av[0], 0.0)
            _zero_vec(den_v, _N)

            def body(i, _):
                s16 = src_v[pl.ds(i * 16, 16)]
                d16 = dst_v[pl.ds(i * 16, 16)]
                a = plsc.load_gather(as_v, [s16])
                b = plsc.load_gather(ad_v, [d16])
                z = a + b
                z = jnp.where(z >= 0.0, z, z * 0.2)
                ex = jnp.exp(z - m)
                ex_v[pl.ds(i * 16, 16)] = ex
                plsc.addupdate_scatter(den_v, [d16], ex)
                return 0
            lax.fori_loop(0, _EPT // 16, body, 0)
            pltpu.sync_copy(ex_v, ex_hbm.at[h, pl.ds(base, _EPT)])
            for g in range(_G):
                pltpu.sync_copy(den_v.at[pl.ds(g * _BN, _BN)],
                                denp_hbm.at[g, wid, h])
    return _sc_gat_edge


def _make_gat_weighted(H):
    seg = _D // (16 * H)

    @functools.partial(
        pl.kernel,
        out_type=jax.ShapeDtypeStruct((_NC, _N, _D), _f32),
        mesh=_mesh,
        compiler_params=pltpu.CompilerParams(needs_layout_passes=False, use_tc_tiling_on_sc=False),
        scratch_types=[
            pltpu.VMEM((_CH,), jnp.int32),
            pltpu.VMEM((_CH,), jnp.int32),
            pltpu.VMEM((_CH, _D), _f32),
            pltpu.VMEM((H, _CH), _f32),
            pltpu.VMEM((125, _D), _f32),
            pltpu.VMEM_SHARED((_N, _D), _f32),
            pltpu.SemaphoreType.DMA,
        ],
    )
    def _sc_gat_agg(src_hbm, dst_hbm, tab_hbm, ex_hbm, out_hbm,
                    sidx, didx, rows, exw, zbuf, acc, sem):
        c, s, wid = _worker_ids()
        base = wid * _EPT
        _zero_acc(zbuf, acc, s)
        plsc.subcore_barrier()

        def body(i, _):
            eb = base + i * _CH
            pltpu.sync_copy(src_hbm.at[pl.ds(eb, _CH)], sidx)
            pltpu.sync_copy(dst_hbm.at[pl.ds(eb, _CH)], didx)
            for h in range(H):
                pltpu.sync_copy(ex_hbm.at[h, pl.ds(eb, _CH)], exw.at[h])
            pltpu.async_copy(tab_hbm.at[sidx], rows, sem).wait()

            def scale(j, _):
                eb16 = j * 16
                wv = [exw[h, pl.ds(eb16, 16)] for h in range(H)]
                for k in range(16):
                    e = eb16 + k
                    for h in range(H):
                        w = wv[h][k]
                        for kk in range(seg):
                            off = h * (_D // H) + kk * 16
                            rows[e, pl.ds(off, 16)] = (
                                rows[e, pl.ds(off, 16)] * w)
                return 0
            lax.fori_loop(0, _CH // 16, scale, 0)
            pltpu.sync_copy(rows, acc.at[didx], add=True)
            return 0
        lax.fori_loop(0, _NCHUNK, body, 0)
        plsc.subcore_barrier()
        rbase = s * _RPT
        pltpu.sync_copy(acc.at[pl.ds(rbase, _RPT)],
                        out_hbm.at[c, pl.ds(rbase, _RPT)])
    return _sc_gat_agg


_gat_edge8 = _make_gat_edge(8)
_gat_edge1 = _make_gat_edge(1)
_gat_agg8 = _make_gat_weighted(8)
_gat_agg1 = _make_gat_weighted(1)


def _cnt_dinv(cntp):
    cnt = jnp.sum(cntp[0], axis=0)
    dinv = lax.rsqrt(cnt + 1.0)[:, None]
    return cnt, dinv


def _leaky(z):
    return jnp.where(z >= 0.0, z, 0.2 * z)


def _tc_k1_body(x_ref, cntp_ref, wgcn_ref, wgat_ref, as_ref, ad_ref,
                g0_ref, hg0_ref, asrcT_ref, adstT_ref, sm_ref, am_ref):
    i = pl.program_id(0)
    _, dinv = _cnt_dinv(cntp_ref[...])
    x = x_ref[...]
    g0_ref[...] = jnp.dot(x, wgcn_ref[...],
                          preferred_element_type=_f32) * dinv
    hg = jnp.dot(x, wgat_ref[...], preferred_element_type=_f32)
    hg0_ref[...] = hg
    h3 = hg.reshape(_BN, 8, 16)
    asrc = jnp.sum(h3 * as_ref[...][None], axis=-1)
    adst = jnp.sum(h3 * ad_ref[...][None], axis=-1)
    asrcT_ref[0] = asrc.T
    adstT_ref[0] = adst.T

    @pl.when(i == 0)
    def _():
        sm_ref[...] = jnp.full((8, 128), -jnp.inf, _f32)
        am_ref[...] = jnp.full((8, 128), -jnp.inf, _f32)
    sm_cur = jnp.broadcast_to(jnp.max(asrc, axis=0)[:, None], (8, 128))
    am_cur = jnp.broadcast_to(jnp.max(adst, axis=0)[:, None], (8, 128))
    sm_ref[...] = jnp.maximum(sm_ref[...], sm_cur)
    am_ref[...] = jnp.maximum(am_ref[...], am_cur)


def _tc_k1(x, cntp, gcn_W0, gat_W0, gat_as0, gat_ad0):
    bs = [
        pl.BlockSpec((_BN, _D), lambda i: (i, 0)),
        pl.BlockSpec((1, _NW, _BN), lambda i: (i, 0, 0)),
        pl.BlockSpec((_D, _D), lambda i: (0, 0)),
        pl.BlockSpec((_D, _D), lambda i: (0, 0)),
        pl.BlockSpec((8, 16), lambda i: (0, 0)),
        pl.BlockSpec((8, 16), lambda i: (0, 0)),
    ]
    outs = [
        jax.ShapeDtypeStruct((_N, _D), _f32),
        jax.ShapeDtypeStruct((_N, _D), _f32),
        jax.ShapeDtypeStruct((_G, 8, _BN), _f32),
        jax.ShapeDtypeStruct((_G, 8, _BN), _f32),
        jax.ShapeDtypeStruct((8, 128), _f32),
        jax.ShapeDtypeStruct((8, 128), _f32),
    ]
    obs = [
        pl.BlockSpec((_BN, _D), lambda i: (i, 0)),
        pl.BlockSpec((_BN, _D), lambda i: (i, 0)),
        pl.BlockSpec((1, 8, _BN), lambda i: (i, 0, 0)),
        pl.BlockSpec((1, 8, _BN), lambda i: (i, 0, 0)),
        pl.BlockSpec((8, 128), lambda i: (0, 0)),
        pl.BlockSpec((8, 128), lambda i: (0, 0)),
    ]
    return pl.pallas_call(
        _tc_k1_body, grid=(_G,), in_specs=bs, out_specs=obs,
        out_shape=outs,
        compiler_params=pltpu.CompilerParams(
            dimension_semantics=("arbitrary",)),
    )(x, cntp, gcn_W0, gat_W0, gat_as0, gat_ad0)


def _tc_k2_body(x_ref, cntp_ref, g0_ref, gP0_ref, ssxP_ref, aggP0_ref,
                denP0_ref, hg0_ref, asrcT0_ref, adstT0_ref, sm0_ref, am0_ref,
                bgcn0_ref, wgcn1_ref, wn0_ref, wr0_ref, bs0_ref,
                bgat0_ref, wgat1_ref, as1_ref, ad1_ref,
                g1_ref, hsage_ref, hg1_ref, asrc1T_ref, adst1T_ref,
                sm1_ref, am1_ref):
    i = pl.program_id(0)
    cnt, dinv = _cnt_dinv(cntp_ref[...])
    x = x_ref[...]
    agg0 = jnp.sum(gP0_ref[...], axis=0) + g0_ref[...]
    hgcn = jnp.maximum(agg0 * dinv + bgcn0_ref[...], 0.0)
    g1_ref[...] = jnp.dot(hgcn, wgcn1_ref[...],
                          preferred_element_type=_f32) * dinv
    mean = jnp.sum(ssxP_ref[...], axis=0) / jnp.maximum(cnt, 1.0)[:, None]
    hsage_ref[...] = jnp.maximum(
        jnp.dot(mean, wn0_ref[...], preferred_element_type=_f32)
        + jnp.dot(x, wr0_ref[...], preferred_element_type=_f32)
        + bs0_ref[...], 0.0)
    m0 = jnp.maximum(sm0_ref[:, :1] + am0_ref[:, :1], 0.0)
    zself = _leaky(asrcT0_ref[0] + adstT0_ref[0])
    exself = jnp.exp(zself - m0)
    den = jnp.sum(denP0_ref[0], axis=0) + exself
    exselfT = exself.T
    deninvT = (1.0 / den).T
    num = (jnp.sum(aggP0_ref[...], axis=0).reshape(_BN, 8, 16)
           + hg0_ref[...].reshape(_BN, 8, 16) * exselfT[:, :, None])
    hgat = jnp.maximum(
        (num * deninvT[:, :, None]).reshape(_BN, _D) + bgat0_ref[...], 0.0)
    hg1 = jnp.dot(hgat, wgat1_ref[...], preferred_element_type=_f32)
    hg1_ref[...] = hg1
    asrc1 = jnp.sum(hg1 * as1_ref[...], axis=1)
    adst1 = jnp.sum(hg1 * ad1_ref[...], axis=1)
    asrc1T_ref[0] = asrc1[None, :]
    adst1T_ref[0] = adst1[None, :]

    @pl.when(i == 0)
    def _():
        sm1_ref[...] = jnp.full((1, 128), -jnp.inf, _f32)
        am1_ref[...] = jnp.full((1, 128), -jnp.inf, _f32)
    sm1_ref[...] = jnp.maximum(sm1_ref[...],
                               jnp.broadcast_to(jnp.max(asrc1), (1, 128)))
    am1_ref[...] = jnp.maximum(am1_ref[...],
                               jnp.broadcast_to(jnp.max(adst1), (1, 128)))


def _tc_k2(x, cntp, g0, gP0, ssxP, aggP0, denP0, hg0, asrcT0, adstT0,
           SM0, AD0, bgcn0, wgcn1, wn0, wr0, bs0, bgat0, wgat1, as1, ad1):
    nd = pl.BlockSpec((_BN, _D), lambda i: (i, 0))
    full = lambda shape: pl.BlockSpec(shape, lambda i: tuple(0 for _ in shape))
    bs = [
        nd,
        pl.BlockSpec((1, _NW, _BN), lambda i: (i, 0, 0)),
        nd,
        pl.BlockSpec((_NC, _BN, _D), lambda i: (0, i, 0)),
        pl.BlockSpec((_NC, _BN, _D), lambda i: (0, i, 0)),
        pl.BlockSpec((_NC, _BN, _D), lambda i: (0, i, 0)),
        pl.BlockSpec((1, _NW, 8, _BN), lambda i: (i, 0, 0, 0)),
        nd,
        pl.BlockSpec((1, 8, _BN), lambda i: (i, 0, 0)),
        pl.BlockSpec((1, 8, _BN), lambda i: (i, 0, 0)),
        full((8, 128)), full((8, 128)),
        full((1, _D)),
        full((_D, _D)),
        full((_D, _D)), full((_D, _D)), full((1, _D)),
        full((1, _D)),
        full((_D, _D)),
        full((1, _D)), full((1, _D)),
    ]
    outs = [
        jax.ShapeDtypeStruct((_N, _D), _f32),
        jax.ShapeDtypeStruct((_N, _D), _f32),
        jax.ShapeDtypeStruct((_N, _D), _f32),
        jax.ShapeDtypeStruct((_G, 1, _BN), _f32),
        jax.ShapeDtypeStruct((_G, 1, _BN), _f32),
        jax.ShapeDtypeStruct((1, 128), _f32),
        jax.ShapeDtypeStruct((1, 128), _f32),
    ]
    obs = [
        nd, nd, nd,
        pl.BlockSpec((1, 1, _BN), lambda i: (i, 0, 0)),
        pl.BlockSpec((1, 1, _BN), lambda i: (i, 0, 0)),
        full((1, 128)), full((1, 128)),
    ]
    return pl.pallas_call(
        _tc_k2_body, grid=(_G,), in_specs=bs, out_specs=obs,
        out_shape=outs,
        compiler_params=pltpu.CompilerParams(
            dimension_semantics=("arbitrary",)),
    )(x, cntp, g0, gP0, ssxP, aggP0, denP0, hg0, asrcT0, adstT0,
      SM0, AD0, bgcn0, wgcn1, wn0, wr0, bs0, bgat0, wgat1, as1, ad1)


def _tc_k3_body(cntp_ref, g1_ref, gP1_ref, hsage_ref, ssageP_ref,
                hg1_ref, aggP1_ref, denP1_ref, asrc1T_ref, adst1T_ref,
                sm1_ref, am1_ref,
                bgcn1_ref, wn1_ref, wr1_ref, bs1_ref, bgat1_ref, out_ref):
    cnt, dinv = _cnt_dinv(cntp_ref[...])
    agg1 = jnp.sum(gP1_ref[...], axis=0) + g1_ref[...]
    c0 = jnp.maximum(agg1 * dinv + bgcn1_ref[...], 0.0)
    mean1 = jnp.sum(ssageP_ref[...], axis=0) / jnp.maximum(cnt, 1.0)[:, None]
    c2 = jnp.maximum(
        jnp.dot(mean1, wn1_ref[...], preferred_element_type=_f32)
        + jnp.dot(hsage_ref[...], wr1_ref[...], preferred_element_type=_f32)
        + bs1_ref[...], 0.0)
    m1 = jnp.maximum(sm1_ref[:, :1] + am1_ref[:, :1], 0.0)
    zself = _leaky(asrc1T_ref[0] + adst1T_ref[0])
    exself = jnp.exp(zself - m1)
    den = jnp.sum(denP1_ref[0], axis=0) + exself
    num = jnp.sum(aggP1_ref[...], axis=0) + hg1_ref[...] * exself.T
    c1 = jnp.maximum(num * (1.0 / den).T + bgat1_ref[...], 0.0)
    out_ref[0] = c0
    out_ref[1] = c1
    out_ref[2] = c2


def _tc_k3(cntp, g1, gP1, hsage, ssageP, hg1, aggP1, denP1, asrc1T, adst1T,
           SM1, AD1, bgcn1, wn1, wr1, bs1, bgat1):
    nd = pl.BlockSpec((_BN, _D), lambda i: (i, 0))
    full = lambda shape: pl.BlockSpec(shape, lambda i: tuple(0 for _ in shape))
    bs = [
        pl.BlockSpec((1, _NW, _BN), lambda i: (i, 0, 0)),
        nd,
        pl.BlockSpec((_NC, _BN, _D), lambda i: (0, i, 0)),
        nd,
        pl.BlockSpec((_NC, _BN, _D), lambda i: (0, i, 0)),
        nd,
        pl.BlockSpec((_NC, _BN, _D), lambda i: (0, i, 0)),
        pl.BlockSpec((1, _NW, 1, _BN), lambda i: (i, 0, 0, 0)),
        pl.BlockSpec((1, 1, _BN), lambda i: (i, 0, 0)),
        pl.BlockSpec((1, 1, _BN), lambda i: (i, 0, 0)),
        full((1, 128)), full((1, 128)),
        full((1, _D)),
        full((_D, _D)), full((_D, _D)), full((1, _D)),
        full((1, _D)),
    ]
    out = jax.ShapeDtypeStruct((3, _N, _D), _f32)
    ob = pl.BlockSpec((3, _BN, _D), lambda i: (0, i, 0))
    return pl.pallas_call(
        _tc_k3_body, grid=(_G,), in_specs=bs, out_specs=ob, out_shape=out,
        compiler_params=pltpu.CompilerParams(
            dimension_semantics=("arbitrary",)),
    )(cntp, g1, gP1, hsage, ssageP, hg1, aggP1, denP1, asrc1T, adst1T,
      SM1, AD1, bgcn1, wn1, wr1, bs1, bgat1)


def kernel(x, edge, batch, gcn_W0, gcn_b0, gcn_W1, gcn_b1,
           sage_Wn0, sage_Wr0, sage_b0, sage_Wn1, sage_Wr1, sage_b1,
           gat_W0, gat_as0, gat_ad0, gat_b0,
           gat_W1, gat_as1, gat_ad1, gat_b1):
    edge = edge.astype(jnp.int32)
    src, dst = edge[0], edge[1]
    bgcn0 = gcn_b0.reshape(1, _D)
    bgcn1 = gcn_b1.reshape(1, _D)
    bs0 = sage_b0.reshape(1, _D)
    bs1 = sage_b1.reshape(1, _D)
    bgat0 = gat_b0.reshape(1, _D)
    bgat1 = gat_b1.reshape(1, _D)
    as1 = gat_as1.reshape(1, _D)
    ad1 = gat_ad1.reshape(1, _D)

    cntp = _sc_degree(dst)
    g0, hg0, asrcT0, adstT0, SM0, AD0 = _tc_k1(
        x, cntp, gcn_W0, gat_W0, gat_as0, gat_ad0)
    gP0 = _sc_segsum(src, dst, g0)
    ssxP = _sc_segsum(src, dst, x)
    ex0, denP0 = _gat_edge8(src, dst, asrcT0, adstT0, SM0, AD0)
    aggP0 = _gat_agg8(src, dst, hg0, ex0)
    g1, hsage, hg1, asrc1T, adst1T, SM1, AD1 = _tc_k2(
        x, cntp, g0, gP0, ssxP, aggP0, denP0, hg0, asrcT0, adstT0,
        SM0, AD0, bgcn0, gcn_W1, sage_Wn0, sage_Wr0, bs0, bgat0,
        gat_W1, as1, ad1)
    gP1 = _sc_segsum(src, dst, g1)
    ssageP = _sc_segsum(src, dst, hsage)
    ex1, denP1 = _gat_edge1(src, dst, asrc1T, adst1T, SM1, AD1)
    aggP1 = _gat_agg1(src, dst, hg1, ex1)
    return _tc_k3(cntp, g1, gP1, hsage, ssageP, hg1, aggP1, denP1,
                  asrc1T, adst1T, SM1, AD1, bgcn1, sage_Wn1, sage_Wr1,
                  bs1, bgat1)


# speedup vs baseline: 31.2571x; 1.4089x over previous
import functools

import jax
import jax.numpy as jnp
from jax import lax
from jax.experimental import pallas as pl
from jax.experimental.pallas import tpu as pltpu
from jax.experimental.pallas import tpu_sc as plsc

_N = 10000
_E = 320000
_D = 128
_NC = 2
_NS = 16
_NW = _NC * _NS
_EPT = _E // _NW
_CH = 80
_NCHUNK = _EPT // _CH
_RPT = _N // _NS
_BN = 1000
_G = _N // _BN

_mesh = plsc.VectorSubcoreMesh(core_axis_name="c", subcore_axis_name="s")
_f32 = jnp.float32


def _worker_ids():
    c = lax.axis_index("c")
    s = lax.axis_index("s")
    return c, s, s * _NC + c


def _zero_vec(ref, n):
    def body(i, _):
        ref[pl.ds(i * 16, 16)] = jnp.zeros((16,), _f32)
        return 0
    lax.fori_loop(0, n // 16, body, 0)


def _zero_acc(zbuf, acc, s):
    def body(i, _):
        for k in range(_D // 16):
            zbuf[i, pl.ds(k * 16, 16)] = jnp.zeros((16,), _f32)
        return 0
    lax.fori_loop(0, 125, body, 0)
    rbase = s * _RPT
    for k in range(5):
        pltpu.sync_copy(zbuf, acc.at[pl.ds(rbase + k * 125, 125)])


@functools.partial(
    pl.kernel,
    out_type=jax.ShapeDtypeStruct((_G, _NW, _BN), _f32),
    mesh=_mesh,
    compiler_params=pltpu.CompilerParams(needs_layout_passes=False, use_tc_tiling_on_sc=False),
    scratch_types=[
        pltpu.VMEM((_EPT,), jnp.int32),
        pltpu.VMEM((_N,), _f32),
    ],
)
def _sc_degree(dst_hbm, cntp_hbm, dst_v, cnt_v):
    _, _, wid = _worker_ids()
    base = wid * _EPT
    pltpu.sync_copy(dst_hbm.at[pl.ds(base, _EPT)], dst_v)
    _zero_vec(cnt_v, _N)
    ones = jnp.ones((16,), _f32)

    def body(i, _):
        d16 = dst_v[pl.ds(i * 16, 16)]
        plsc.addupdate_scatter(cnt_v, [d16], ones)
        return 0
    lax.fori_loop(0, _EPT // 16, body, 0)
    for g in range(_G):
        pltpu.sync_copy(cnt_v.at[pl.ds(g * _BN, _BN)], cntp_hbm.at[g, wid])


@functools.partial(
    pl.kernel,
    out_type=jax.ShapeDtypeStruct((_NC, _N, _D), _f32),
    mesh=_mesh,
    compiler_params=pltpu.CompilerParams(needs_layout_passes=False, use_tc_tiling_on_sc=False),
    scratch_types=[
        pltpu.VMEM((_NCHUNK, _CH), jnp.int32),
        pltpu.VMEM((_NCHUNK, _CH), jnp.int32),
        pltpu.VMEM((_CH, _D), _f32),
        pltpu.VMEM((125, _D), _f32),
        pltpu.VMEM_SHARED((_N, _D), _f32),
        pltpu.SemaphoreType.DMA,
    ],
)
def _sc_segsum(src_hbm, dst_hbm, tab_hbm, out_hbm, sidx, didx, rows, zbuf,
               acc, sem):
    c, s, wid = _worker_ids()
    pltpu.sync_copy(src_hbm.at[wid], sidx)
    pltpu.sync_copy(dst_hbm.at[wid], didx)
    _zero_acc(zbuf, acc, s)
    plsc.subcore_barrier()

    def body(i, _):
        pltpu.async_copy(tab_hbm.at[sidx.at[i]], rows, sem).wait()
        pltpu.sync_copy(rows, acc.at[didx.at[i]], add=True)
        return 0
    lax.fori_loop(0, _NCHUNK, body, 0)
    plsc.subcore_barrier()
    rbase = s * _RPT
    pltpu.sync_copy(acc.at[pl.ds(rbase, _RPT)],
                    out_hbm.at[c, pl.ds(rbase, _RPT)])


@functools.partial(
    pl.kernel,
    out_type=(jax.ShapeDtypeStruct((_NW, _NCHUNK, 8, _CH), _f32),
              jax.ShapeDtypeStruct((_NW, _N * 8), _f32)),
    mesh=_mesh,
    compiler_params=pltpu.CompilerParams(needs_layout_passes=False, use_tc_tiling_on_sc=False),
    scratch_types=[
        pltpu.VMEM((_NCHUNK, _CH), jnp.int32),
        pltpu.VMEM((_NCHUNK, _CH), jnp.int32),
        pltpu.VMEM((_CH, 8), _f32),
        pltpu.VMEM((_CH, 8), _f32),
        pltpu.VMEM((8, _CH), _f32),
        pltpu.VMEM((_N * 8,), _f32),
        pltpu.VMEM((8, 128), _f32),
        pltpu.VMEM((8, 128), _f32),
        pltpu.SemaphoreType.DMA,
    ],
)
def _gat_edge8(src_hbm, dst_hbm, asrc_hbm, adst_hbm, sm_hbm, am_hbm,
               ex_hbm, denp_hbm,
               sidx, didx, arows, brows, exb, den_v, sm_v, am_v, sem):
    _, _, wid = _worker_ids()
    pltpu.sync_copy(src_hbm.at[wid], sidx)
    pltpu.sync_copy(dst_hbm.at[wid], didx)
    pltpu.sync_copy(sm_hbm, sm_v)
    pltpu.sync_copy(am_hbm, am_v)
    ms = []
    for h in range(8):
        sv = sm_v[h, pl.ds(0, 16)]
        av = am_v[h, pl.ds(0, 16)]
        ms.append(jnp.maximum(sv[0] + av[0], 0.0))
    _zero_vec(den_v, _N * 8)
    iota16 = lax.iota(jnp.int32, 16)

    def body(j, _):
        d1 = pltpu.async_copy(asrc_hbm.at[sidx.at[j]], arows, sem)
        d2 = pltpu.async_copy(adst_hbm.at[didx.at[j]], brows, sem)
        d1.wait()
        d2.wait()
        for off in range(0, _CH, 16):
            e16 = iota16 + off
            d16 = didx[j, pl.ds(off, 16)]
            d16x8 = d16 * 8
            for h in range(8):
                hh = jnp.full((16,), h, jnp.int32)
                a = plsc.load_gather(arows, [e16, hh])
                b = plsc.load_gather(brows, [e16, hh])
                z = a + b
                z = jnp.where(z >= 0.0, z, z * 0.2)
                ex = jnp.exp(z - ms[h])
                exb[h, pl.ds(off, 16)] = ex
                plsc.addupdate_scatter(den_v, [d16x8 + h], ex)
        pltpu.sync_copy(exb, ex_hbm.at[wid, j])
        return 0
    lax.fori_loop(0, _NCHUNK, body, 0)
    pltpu.sync_copy(den_v, denp_hbm.at[wid])


@functools.partial(
    pl.kernel,
    out_type=(jax.ShapeDtypeStruct((_NW, _NCHUNK, _CH), _f32),
              jax.ShapeDtypeStruct((_NW, _N), _f32)),
    mesh=_mesh,
    compiler_params=pltpu.CompilerParams(needs_layout_passes=False, use_tc_tiling_on_sc=False),
    scratch_types=[
        pltpu.VMEM((_NCHUNK, _CH), jnp.int32),
        pltpu.VMEM((_NCHUNK, _CH), jnp.int32),
        pltpu.VMEM((_N,), _f32),
        pltpu.VMEM((_N,), _f32),
        pltpu.VMEM((_CH,), _f32),
        pltpu.VMEM((_N,), _f32),
        pltpu.VMEM((1, 128), _f32),
        pltpu.VMEM((1, 128), _f32),
    ],
)
def _gat_edge1(src_hbm, dst_hbm, asrc_hbm, adst_hbm, sm_hbm, am_hbm,
               ex_hbm, denp_hbm,
               sidx, didx, as_v, ad_v, exb, den_v, sm_v, am_v):
    _, _, wid = _worker_ids()
    pltpu.sync_copy(src_hbm.at[wid], sidx)
    pltpu.sync_copy(dst_hbm.at[wid], didx)
    pltpu.sync_copy(asrc_hbm, as_v)
    pltpu.sync_copy(adst_hbm, ad_v)
    pltpu.sync_copy(sm_hbm, sm_v)
    pltpu.sync_copy(am_hbm, am_v)
    sv = sm_v[0, pl.ds(0, 16)]
    av = am_v[0, pl.ds(0, 16)]
    m = jnp.maximum(sv[0] + av[0], 0.0)
    _zero_vec(den_v, _N)

    def body(j, _):
        for off in range(0, _CH, 16):
            s16 = sidx[j, pl.ds(off, 16)]
            d16 = didx[j, pl.ds(off, 16)]
            a = plsc.load_gather(as_v, [s16])
            b = plsc.load_gather(ad_v, [d16])
            z = a + b
            z = jnp.where(z >= 0.0, z, z * 0.2)
            ex = jnp.exp(z - m)
            exb[pl.ds(off, 16)] = ex
            plsc.addupdate_scatter(den_v, [d16], ex)
        pltpu.sync_copy(exb, ex_hbm.at[wid, j])
        return 0
    lax.fori_loop(0, _NCHUNK, body, 0)
    pltpu.sync_copy(den_v, denp_hbm.at[wid])


def _make_gat_weighted(H):
    seg = _D // (16 * H)
    ex_shape = (_NW, _NCHUNK, 8, _CH) if H == 8 else (_NW, _NCHUNK, _CH)
    exw_shape = (8, _CH) if H == 8 else (_CH,)

    @functools.partial(
        pl.kernel,
        out_type=jax.ShapeDtypeStruct((_NC, _N, _D), _f32),
        mesh=_mesh,
        compiler_params=pltpu.CompilerParams(needs_layout_passes=False, use_tc_tiling_on_sc=False),
        scratch_types=[
            pltpu.VMEM((_NCHUNK, _CH), jnp.int32),
            pltpu.VMEM((_NCHUNK, _CH), jnp.int32),
            pltpu.VMEM((_CH, _D), _f32),
            pltpu.VMEM(exw_shape, _f32),
            pltpu.VMEM((125, _D), _f32),
            pltpu.VMEM_SHARED((_N, _D), _f32),
            pltpu.SemaphoreType.DMA,
        ],
    )
    def _sc_gat_agg(src_hbm, dst_hbm, tab_hbm, ex_hbm, out_hbm,
                    sidx, didx, rows, exw, zbuf, acc, sem):
        c, s, wid = _worker_ids()
        pltpu.sync_copy(src_hbm.at[wid], sidx)
        pltpu.sync_copy(dst_hbm.at[wid], didx)
        _zero_acc(zbuf, acc, s)
        plsc.subcore_barrier()

        def body(i, _):
            pltpu.sync_copy(ex_hbm.at[wid, i], exw)
            pltpu.async_copy(tab_hbm.at[sidx.at[i]], rows, sem).wait()

            def scale(j, _):
                eb16 = j * 16
                if H == 8:
                    wv = [exw[h, pl.ds(eb16, 16)] for h in range(H)]
                else:
                    wv = [exw[pl.ds(eb16, 16)]]
                for k in range(16):
                    e = eb16 + k
                    for h in range(H):
                        w = wv[h][k]
                        for kk in range(seg):
                            off = h * (_D // H) + kk * 16
                            rows[e, pl.ds(off, 16)] = (
                                rows[e, pl.ds(off, 16)] * w)
                return 0
            lax.fori_loop(0, _CH // 16, scale, 0)
            pltpu.sync_copy(rows, acc.at[didx.at[i]], add=True)
            return 0
        lax.fori_loop(0, _NCHUNK, body, 0)
        plsc.subcore_barrier()
        rbase = s * _RPT
        pltpu.sync_copy(acc.at[pl.ds(rbase, _RPT)],
                        out_hbm.at[c, pl.ds(rbase, _RPT)])
    return _sc_gat_agg


_gat_agg8 = _make_gat_weighted(8)
_gat_agg1 = _make_gat_weighted(1)


def _cnt_dinv(cntp):
    cnt = jnp.sum(cntp[0], axis=0)
    dinv = lax.rsqrt(cnt + 1.0)[:, None]
    return cnt, dinv


def _leaky(z):
    return jnp.where(z >= 0.0, z, 0.2 * z)


def _tc_k1_body(x_ref, cntp_ref, wgcn_ref, wgat_ref, as_ref, ad_ref,
                g0_ref, hg0_ref, asrcT_ref, adstT_ref, sm_ref, am_ref,
                smr_ref, amr_ref):
    i = pl.program_id(0)
    _, dinv = _cnt_dinv(cntp_ref[...])
    x = x_ref[...]
    g0_ref[...] = jnp.dot(x, wgcn_ref[...],
                          preferred_element_type=_f32) * dinv
    hg = jnp.dot(x, wgat_ref[...], preferred_element_type=_f32)
    hg0_ref[...] = hg
    h3 = hg.reshape(_BN, 8, 16)
    asrc = jnp.sum(h3 * as_ref[...][None], axis=-1)
    adst = jnp.sum(h3 * ad_ref[...][None], axis=-1)
    asrcT_ref[...] = asrc
    adstT_ref[...] = adst

    @pl.when(i == 0)
    def _():
        sm_ref[...] = jnp.full((8, 128), -jnp.inf, _f32)
        am_ref[...] = jnp.full((8, 128), -jnp.inf, _f32)
        smr_ref[...] = jnp.full((1, 8), -jnp.inf, _f32)
        amr_ref[...] = jnp.full((1, 8), -jnp.inf, _f32)
    amax = jnp.max(asrc, axis=0)
    bmax = jnp.max(adst, axis=0)
    sm_cur = jnp.broadcast_to(amax[:, None], (8, 128))
    am_cur = jnp.broadcast_to(bmax[:, None], (8, 128))
    sm_ref[...] = jnp.maximum(sm_ref[...], sm_cur)
    am_ref[...] = jnp.maximum(am_ref[...], am_cur)
    smr_ref[...] = jnp.maximum(smr_ref[...], amax[None, :])
    amr_ref[...] = jnp.maximum(amr_ref[...], bmax[None, :])


def _tc_k1(x, cntp, gcn_W0, gat_W0, gat_as0, gat_ad0):
    bs = [
        pl.BlockSpec((_BN, _D), lambda i: (i, 0)),
        pl.BlockSpec((1, _NW, _BN), lambda i: (i, 0, 0)),
        pl.BlockSpec((_D, _D), lambda i: (0, 0)),
        pl.BlockSpec((_D, _D), lambda i: (0, 0)),
        pl.BlockSpec((8, 16), lambda i: (0, 0)),
        pl.BlockSpec((8, 16), lambda i: (0, 0)),
    ]
    outs = [
        jax.ShapeDtypeStruct((_N, _D), _f32),
        jax.ShapeDtypeStruct((_N, _D), _f32),
        jax.ShapeDtypeStruct((_N, 8), _f32),
        jax.ShapeDtypeStruct((_N, 8), _f32),
        jax.ShapeDtypeStruct((8, 128), _f32),
        jax.ShapeDtypeStruct((8, 128), _f32),
        jax.ShapeDtypeStruct((1, 8), _f32),
        jax.ShapeDtypeStruct((1, 8), _f32),
    ]
    obs = [
        pl.BlockSpec((_BN, _D), lambda i: (i, 0)),
        pl.BlockSpec((_BN, _D), lambda i: (i, 0)),
        pl.BlockSpec((_BN, 8), lambda i: (i, 0)),
        pl.BlockSpec((_BN, 8), lambda i: (i, 0)),
        pl.BlockSpec((8, 128), lambda i: (0, 0)),
        pl.BlockSpec((8, 128), lambda i: (0, 0)),
        pl.BlockSpec((1, 8), lambda i: (0, 0)),
        pl.BlockSpec((1, 8), lambda i: (0, 0)),
    ]
    return pl.pallas_call(
        _tc_k1_body, grid=(_G,), in_specs=bs, out_specs=obs,
        out_shape=outs,
        compiler_params=pltpu.CompilerParams(
            dimension_semantics=("arbitrary",)),
    )(x, cntp, gcn_W0, gat_W0, gat_as0, gat_ad0)


def _tc_k2_body(x_ref, cntp_ref, g0_ref, gP0_ref, ssxP_ref, aggP0_ref,
                denP0_ref, hg0_ref, asrcT0_ref, adstT0_ref, sm0_ref, am0_ref,
                bgcn0_ref, wgcn1_ref, wn0_ref, wr0_ref, bs0_ref,
                bgat0_ref, wgat1_ref, as1_ref, ad1_ref,
                g1_ref, hsage_ref, hg1_ref, asrc1T_ref, adst1T_ref,
                sm1_ref, am1_ref):
    i = pl.program_id(0)
    cnt, dinv = _cnt_dinv(cntp_ref[...])
    x = x_ref[...]
    agg0 = jnp.sum(gP0_ref[...], axis=0) + g0_ref[...]
    hgcn = jnp.maximum(agg0 * dinv + bgcn0_ref[...], 0.0)
    g1_ref[...] = jnp.dot(hgcn, wgcn1_ref[...],
                          preferred_element_type=_f32) * dinv
    mean = jnp.sum(ssxP_ref[...], axis=0) / jnp.maximum(cnt, 1.0)[:, None]
    hsage_ref[...] = jnp.maximum(
        jnp.dot(mean, wn0_ref[...], preferred_element_type=_f32)
        + jnp.dot(x, wr0_ref[...], preferred_element_type=_f32)
        + bs0_ref[...], 0.0)
    m0 = jnp.maximum(sm0_ref[...] + am0_ref[...], 0.0)
    zself = _leaky(asrcT0_ref[...] + adstT0_ref[...])
    exself = jnp.exp(zself - m0)
    den = jnp.sum(denP0_ref[...], axis=0) + exself
    deninv = 1.0 / den
    num = (jnp.sum(aggP0_ref[...], axis=0).reshape(_BN, 8, 16)
           + hg0_ref[...].reshape(_BN, 8, 16) * exself[:, :, None])
    hgat = jnp.maximum(
        (num * deninv[:, :, None]).reshape(_BN, _D) + bgat0_ref[...], 0.0)
    hg1 = jnp.dot(hgat, wgat1_ref[...], preferred_element_type=_f32)
    hg1_ref[...] = hg1
    asrc1 = jnp.sum(hg1 * as1_ref[...], axis=1, keepdims=True)
    adst1 = jnp.sum(hg1 * ad1_ref[...], axis=1, keepdims=True)
    asrc1T_ref[...] = asrc1
    adst1T_ref[...] = adst1

    @pl.when(i == 0)
    def _():
        sm1_ref[...] = jnp.full((1, 128), -jnp.inf, _f32)
        am1_ref[...] = jnp.full((1, 128), -jnp.inf, _f32)
    sm1_ref[...] = jnp.maximum(sm1_ref[...],
                               jnp.broadcast_to(jnp.max(asrc1), (1, 128)))
    am1_ref[...] = jnp.maximum(am1_ref[...],
                               jnp.broadcast_to(jnp.max(adst1), (1, 128)))


def _tc_k2(x, cntp, g0, gP0, ssxP, aggP0, denP0, hg0, asrcT0, adstT0,
           SM0, AD0, bgcn0, wgcn1, wn0, wr0, bs0, bgat0, wgat1, as1, ad1):
    nd = pl.BlockSpec((_BN, _D), lambda i: (i, 0))
    full = lambda shape: pl.BlockSpec(shape, lambda i: tuple(0 for _ in shape))
    bs = [
        nd,
        pl.BlockSpec((1, _NW, _BN), lambda i: (i, 0, 0)),
        nd,
        pl.BlockSpec((_NC, _BN, _D), lambda i: (0, i, 0)),
        pl.BlockSpec((_NC, _BN, _D), lambda i: (0, i, 0)),
        pl.BlockSpec((_NC, _BN, _D), lambda i: (0, i, 0)),
        pl.BlockSpec((_NW, _BN, 8), lambda i: (0, i, 0)),
        nd,
        pl.BlockSpec((_BN, 8), lambda i: (i, 0)),
        pl.BlockSpec((_BN, 8), lambda i: (i, 0)),
        full((1, 8)), full((1, 8)),
        full((1, _D)),
        full((_D, _D)),
        full((_D, _D)), full((_D, _D)), full((1, _D)),
        full((1, _D)),
        full((_D, _D)),
        full((1, _D)), full((1, _D)),
    ]
    outs = [
        jax.ShapeDtypeStruct((_N, _D), _f32),
        jax.ShapeDtypeStruct((_N, _D), _f32),
        jax.ShapeDtypeStruct((_N, _D), _f32),
        jax.ShapeDtypeStruct((_N, 1), _f32),
        jax.ShapeDtypeStruct((_N, 1), _f32),
        jax.ShapeDtypeStruct((1, 128), _f32),
        jax.ShapeDtypeStruct((1, 128), _f32),
    ]
    obs = [
        nd, nd, nd,
        pl.BlockSpec((_BN, 1), lambda i: (i, 0)),
        pl.BlockSpec((_BN, 1), lambda i: (i, 0)),
        full((1, 128)), full((1, 128)),
    ]
    return pl.pallas_call(
        _tc_k2_body, grid=(_G,), in_specs=bs, out_specs=obs,
        out_shape=outs,
        compiler_params=pltpu.CompilerParams(
            dimension_semantics=("arbitrary",)),
    )(x, cntp, g0, gP0, ssxP, aggP0, denP0, hg0, asrcT0, adstT0,
      SM0, AD0, bgcn0, wgcn1, wn0, wr0, bs0, bgat0, wgat1, as1, ad1)


def _tc_k3_body(cntp_ref, g1_ref, gP1_ref, hsage_ref, ssageP_ref,
                hg1_ref, aggP1_ref, denP1_ref, asrc1T_ref, adst1T_ref,
                sm1_ref, am1_ref,
                bgcn1_ref, wn1_ref, wr1_ref, bs1_ref, bgat1_ref, out_ref):
    cnt, dinv = _cnt_dinv(cntp_ref[...])
    agg1 = jnp.sum(gP1_ref[...], axis=0) + g1_ref[...]
    c0 = jnp.maximum(agg1 * dinv + bgcn1_ref[...], 0.0)
    mean1 = jnp.sum(ssageP_ref[...], axis=0) / jnp.maximum(cnt, 1.0)[:, None]
    c2 = jnp.maximum(
        jnp.dot(mean1, wn1_ref[...], preferred_element_type=_f32)
        + jnp.dot(hsage_ref[...], wr1_ref[...], preferred_element_type=_f32)
        + bs1_ref[...], 0.0)
    m1 = jnp.maximum(sm1_ref[:, :1] + am1_ref[:, :1], 0.0)
    zself = _leaky(asrc1T_ref[...] + adst1T_ref[...])
    exself = jnp.exp(zself - m1)
    den = jnp.sum(denP1_ref[...], axis=0) + exself
    num = jnp.sum(aggP1_ref[...], axis=0) + hg1_ref[...] * exself
    c1 = jnp.maximum(num * (1.0 / den) + bgat1_ref[...], 0.0)
    out_ref[0] = c0
    out_ref[1] = c1
    out_ref[2] = c2


def _tc_k3(cntp, g1, gP1, hsage, ssageP, hg1, aggP1, denP1, asrc1T, adst1T,
           SM1, AD1, bgcn1, wn1, wr1, bs1, bgat1):
    nd = pl.BlockSpec((_BN, _D), lambda i: (i, 0))
    full = lambda shape: pl.BlockSpec(shape, lambda i: tuple(0 for _ in shape))
    bs = [
        pl.BlockSpec((1, _NW, _BN), lambda i: (i, 0, 0)),
        nd,
        pl.BlockSpec((_NC, _BN, _D), lambda i: (0, i, 0)),
        nd,
        pl.BlockSpec((_NC, _BN, _D), lambda i: (0, i, 0)),
        nd,
        pl.BlockSpec((_NC, _BN, _D), lambda i: (0, i, 0)),
        pl.BlockSpec((_NW, _BN, 1), lambda i: (0, i, 0)),
        pl.BlockSpec((_BN, 1), lambda i: (i, 0)),
        pl.BlockSpec((_BN, 1), lambda i: (i, 0)),
        full((1, 128)), full((1, 128)),
        full((1, _D)),
        full((_D, _D)), full((_D, _D)), full((1, _D)),
        full((1, _D)),
    ]
    out = jax.ShapeDtypeStruct((3, _N, _D), _f32)
    ob = pl.BlockSpec((3, _BN, _D), lambda i: (0, i, 0))
    return pl.pallas_call(
        _tc_k3_body, grid=(_G,), in_specs=bs, out_specs=ob, out_shape=out,
        compiler_params=pltpu.CompilerParams(
            dimension_semantics=("arbitrary",)),
    )(cntp, g1, gP1, hsage, ssageP, hg1, aggP1, denP1, asrc1T, adst1T,
      SM1, AD1, bgcn1, wn1, wr1, bs1, bgat1)


def kernel(x, edge, batch, gcn_W0, gcn_b0, gcn_W1, gcn_b1,
           sage_Wn0, sage_Wr0, sage_b0, sage_Wn1, sage_Wr1, sage_b1,
           gat_W0, gat_as0, gat_ad0, gat_b0,
           gat_W1, gat_as1, gat_ad1, gat_b1):
    edge = edge.astype(jnp.int32)
    src, dst = edge[0], edge[1]
    bgcn0 = gcn_b0.reshape(1, _D)
    bgcn1 = gcn_b1.reshape(1, _D)
    bs0 = sage_b0.reshape(1, _D)
    bs1 = sage_b1.reshape(1, _D)
    bgat0 = gat_b0.reshape(1, _D)
    bgat1 = gat_b1.reshape(1, _D)
    as1 = gat_as1.reshape(1, _D)
    ad1 = gat_ad1.reshape(1, _D)

    src3 = src.reshape(_NW, _NCHUNK, _CH)
    dst3 = dst.reshape(_NW, _NCHUNK, _CH)

    cntp = _sc_degree(dst)
    g0, hg0, asrc0, adst0, SM0, AD0, SMr0, ADr0 = _tc_k1(
        x, cntp, gcn_W0, gat_W0, gat_as0, gat_ad0)
    gP0 = _sc_segsum(src3, dst3, g0)
    ssxP = _sc_segsum(src3, dst3, x)
    ex0, denP0f = _gat_edge8(src3, dst3, asrc0, adst0, SM0, AD0)
    denP0 = denP0f.reshape(_NW, _N, 8)
    aggP0 = _gat_agg8(src3, dst3, hg0, ex0)
    g1, hsage, hg1, asrc1, adst1, SM1, AD1 = _tc_k2(
        x, cntp, g0, gP0, ssxP, aggP0, denP0, hg0, asrc0, adst0,
        SMr0, ADr0, bgcn0, gcn_W1, sage_Wn0, sage_Wr0, bs0, bgat0,
        gat_W1, as1, ad1)
    gP1 = _sc_segsum(src3, dst3, g1)
    ssageP = _sc_segsum(src3, dst3, hsage)
    ex1, denP1f = _gat_edge1(src3, dst3, asrc1.reshape(_N),
                             adst1.reshape(_N), SM1, AD1)
    denP1 = denP1f.reshape(_NW, _N, 1)
    aggP1 = _gat_agg1(src3, dst3, hg1, ex1)
    return _tc_k3(cntp, g1, gP1, hsage, ssageP, hg1, aggP1, denP1,
                  asrc1, adst1, SM1, AD1, bgcn1, sage_Wn1, sage_Wr1,
                  bs1, bgat1)
